# Initial kernel scaffold; baseline (speedup 1.0000x reference)
import jax
import jax.numpy as jnp
from jax.experimental import pallas as pl


def kernel(x, edge_index, W0, as0, ad0, b0, W1, as1, ad1, b1, W2, as2, ad2, b2):
    raise NotImplementedError("write your pallas kernel here")


# speedup vs baseline: 1.0068x; 1.0068x over previous
import jax
import jax.numpy as jnp
from jax.experimental import pallas as pl
from jax.experimental.pallas import tpu as pltpu


def _mm_kernel(x_ref, w_ref, o_ref):
    o_ref[...] = jnp.dot(x_ref[...], w_ref[...],
                         preferred_element_type=jnp.float32)


def _matmul(x, w):
    n, k = x.shape
    k2, m = w.shape
    bn = 1000
    return pl.pallas_call(
        _mm_kernel,
        grid=(n // bn,),
        in_specs=[pl.BlockSpec((bn, k), lambda i: (i, 0)),
                  pl.BlockSpec((k, m), lambda i: (0, 0))],
        out_specs=pl.BlockSpec((bn, m), lambda i: (i, 0)),
        out_shape=jax.ShapeDtypeStruct((n, m), jnp.float32),
    )(x, w)


def _gat_conv(x, src, dst, W, att_src, att_dst, bias, heads, out_ch):
    N = x.shape[0]
    h = _matmul(x, W).reshape(N, heads, out_ch)
    a_src = jnp.sum(h * att_src[None, :, :], axis=-1)
    a_dst = jnp.sum(h * att_dst[None, :, :], axis=-1)
    alpha = a_src[src] + a_dst[dst]
    alpha = jax.nn.leaky_relu(alpha, negative_slope=0.2)
    amax = jax.ops.segment_max(alpha, dst, num_segments=N)
    ex = jnp.exp(alpha - amax[dst])
    denom = jax.ops.segment_sum(ex, dst, num_segments=N)
    coef = ex / (denom[dst] + 1e-16)
    msg = h[src] * coef[:, :, None]
    out = jax.ops.segment_sum(msg, dst, num_segments=N)
    out = out.reshape(N, heads * out_ch)
    return out + bias


def kernel(x, edge_index, W0, as0, ad0, b0, W1, as1, ad1, b1, W2, as2, ad2, b2):
    N = x.shape[0]
    loop = jnp.arange(N, dtype=edge_index.dtype)
    src = jnp.concatenate([edge_index[0], loop])
    dst = jnp.concatenate([edge_index[1], loop])
    h = jax.nn.elu(_gat_conv(x, src, dst, W0, as0, ad0, b0, 8, 64))
    h = jax.nn.elu(_gat_conv(h, src, dst, W1, as1, ad1, b1, 8, 64))
    h = _gat_conv(h, src, dst, W2, as2, ad2, b2, 1, 1)
    return jax.nn.sigmoid(h).reshape(-1)


# speedup vs baseline: 8.1539x; 8.0986x over previous
import jax
import jax.numpy as jnp
from jax import lax
from jax.experimental import pallas as pl
from jax.experimental.pallas import tpu as pltpu
from jax.experimental.pallas import tpu_sc as plsc

N = 10000
NP = 10240
NPH = NP // 2
NT = 16
E_REAL = 330000
CH = 168
CS = 24
B = 128
EP = NT * CH * B
NB = NP // 1024
NEG = -1e30


def _attn_tail(h, as_w, ad_w, i, h4_ref, ls_ref, ld_ref, amax_ref):
    hr = h.reshape(1024, 8, 64)
    a_s = jnp.sum(hr * as_w[None], axis=-1)
    a_d = jnp.sum(hr * ad_w[None], axis=-1)
    rid = lax.broadcasted_iota(jnp.int32, (1024, 8), 0) + i * 1024
    a_s = jnp.where(rid >= N, NEG, a_s)
    h4_ref[...] = h.reshape(1024, 4, 128).transpose(1, 0, 2)
    ls_ref[...] = jnp.pad(a_s, ((0, 0), (0, 120)))
    ld_ref[...] = jnp.pad(a_d, ((0, 0), (0, 120)))

    @pl.when(i == 0)
    def _():
        amax_ref[...] = jnp.full((8, 128), NEG, jnp.float32)

    bm = jnp.max(a_s, axis=0)
    amax_ref[...] = jnp.maximum(amax_ref[...], bm[:, None])


def _pre_kernel(x_ref, w_ref, as_ref, ad_ref,
                h4_ref, ls_ref, ld_ref, amax_ref):
    i = pl.program_id(0)
    h = jnp.dot(x_ref[...], w_ref[...], preferred_element_type=jnp.float32)
    _attn_tail(h, as_ref[...], ad_ref[...], i,
               h4_ref, ls_ref, ld_ref, amax_ref)


def _finalize_prev(acc_ref, den_ref, b_ref, i):
    acc = acc_ref[...]
    den = den_ref[...]
    a = acc.transpose(1, 0, 2).reshape(1024, 512)
    dn = den.reshape(8, 1024).T
    hsel = (lax.broadcasted_iota(jnp.int32, (8, 512), 1) // 64 ==
            lax.broadcasted_iota(jnp.int32, (8, 512), 0))
    db = jnp.dot(dn, hsel.astype(jnp.float32),
                 preferred_element_type=jnp.float32)
    y = a / (db + 1e-16) + b_ref[...]
    y = jnp.where(y > 0, y, jnp.exp(jnp.minimum(y, 0.0)) - 1.0)
    rid = lax.broadcasted_iota(jnp.int32, (1024, 512), 0) + i * 1024
    return jnp.where(rid >= N, 0.0, y)


def _mid_kernel(acc_ref, den_ref, b_ref, w_ref, as_ref, ad_ref,
                h4_ref, ls_ref, ld_ref, amax_ref):
    i = pl.program_id(0)
    y = _finalize_prev(acc_ref, den_ref, b_ref, i)
    h = jnp.dot(y, w_ref[...], preferred_element_type=jnp.float32)
    _attn_tail(h, as_ref[...], ad_ref[...], i,
               h4_ref, ls_ref, ld_ref, amax_ref)


def _mid2_kernel(acc_ref, den_ref, b_ref, w_ref, as_ref, ad_ref,
                 lt_ref, amax_ref):
    i = pl.program_id(0)
    y = _finalize_prev(acc_ref, den_ref, b_ref, i)
    h2f = jnp.dot(y, w_ref[...], preferred_element_type=jnp.float32)
    h2 = h2f[:, 0:1]
    asv = h2 * as_ref[0, 0]
    adv = h2 * ad_ref[0, 0]
    rid = lax.broadcasted_iota(jnp.int32, (1024, 1), 0) + i * 1024
    asv = jnp.where(rid >= N, NEG, asv)
    lt_ref[...] = jnp.pad(jnp.concatenate([h2, asv, adv], axis=1),
                          ((0, 0), (0, 125)))

    @pl.when(i == 0)
    def _():
        amax_ref[...] = jnp.full((8, 128), NEG, jnp.float32)

    amax_ref[...] = jnp.maximum(amax_ref[...], jnp.max(asv))


def _fin_kernel(o2_ref, b2_ref, out_ref):
    o = o2_ref[...]
    num = o[0, 0] + o[1, 0]
    den = o[0, 1] + o[1, 1]
    r = num / (den + 1e-16) + b2_ref[0, 0]
    out_ref[...] = jax.nn.sigmoid(r).reshape(1, 8, 128)


def _pre(xp, W0, as0, ad0):
    return pl.pallas_call(
        _pre_kernel,
        grid=(NB,),
        in_specs=[
            pl.BlockSpec((1024, 128), lambda i: (i, 0)),
            pl.BlockSpec((128, 512), lambda i: (0, 0)),
            pl.BlockSpec((8, 64), lambda i: (0, 0)),
            pl.BlockSpec((8, 64), lambda i: (0, 0)),
        ],
        out_specs=[
            pl.BlockSpec((4, 1024, 128), lambda i: (0, i, 0)),
            pl.BlockSpec((1024, 128), lambda i: (i, 0)),
            pl.BlockSpec((1024, 128), lambda i: (i, 0)),
            pl.BlockSpec((8, 128), lambda i: (0, 0)),
        ],
        out_shape=[
            jax.ShapeDtypeStruct((4, NP, 128), jnp.float32),
            jax.ShapeDtypeStruct((NP, 128), jnp.float32),
            jax.ShapeDtypeStruct((NP, 128), jnp.float32),
            jax.ShapeDtypeStruct((8, 128), jnp.float32),
        ],
    )(xp, W0, as0, ad0)


def _mid(acc4, den4, b, W, as_w, ad_w):
    return pl.pallas_call(
        _mid_kernel,
        grid=(NB,),
        in_specs=[
            pl.BlockSpec((4, 1024, 128), lambda i: (0, i, 0)),
            pl.BlockSpec((4, 2, 1024), lambda i: (0, 0, i)),
            pl.BlockSpec((1, 512), lambda i: (0, 0)),
            pl.BlockSpec((512, 512), lambda i: (0, 0)),
            pl.BlockSpec((8, 64), lambda i: (0, 0)),
            pl.BlockSpec((8, 64), lambda i: (0, 0)),
        ],
        out_specs=[
            pl.BlockSpec((4, 1024, 128), lambda i: (0, i, 0)),
            pl.BlockSpec((1024, 128), lambda i: (i, 0)),
            pl.BlockSpec((1024, 128), lambda i: (i, 0)),
            pl.BlockSpec((8, 128), lambda i: (0, 0)),
        ],
        out_shape=[
            jax.ShapeDtypeStruct((4, NP, 128), jnp.float32),
            jax.ShapeDtypeStruct((NP, 128), jnp.float32),
            jax.ShapeDtypeStruct((NP, 128), jnp.float32),
            jax.ShapeDtypeStruct((8, 128), jnp.float32),
        ],
    )(acc4, den4, b, W, as_w, ad_w)


def _mid2(acc4, den4, b, W2p, as2, ad2):
    return pl.pallas_call(
        _mid2_kernel,
        grid=(NB,),
        in_specs=[
            pl.BlockSpec((4, 1024, 128), lambda i: (0, i, 0)),
            pl.BlockSpec((4, 2, 1024), lambda i: (0, 0, i)),
            pl.BlockSpec((1, 512), lambda i: (0, 0)),
            pl.BlockSpec((512, 128), lambda i: (0, 0)),
            pl.BlockSpec(memory_space=pltpu.SMEM),
            pl.BlockSpec(memory_space=pltpu.SMEM),
        ],
        out_specs=[
            pl.BlockSpec((1024, 128), lambda i: (i, 0)),
            pl.BlockSpec((8, 128), lambda i: (0, 0)),
        ],
        out_shape=[
            jax.ShapeDtypeStruct((NP, 128), jnp.float32),
            jax.ShapeDtypeStruct((8, 128), jnp.float32),
        ],
    )(acc4, den4, b, W2p, as2, ad2)


def _fin(o2, b2):
    return pl.pallas_call(
        _fin_kernel,
        grid=(NB,),
        in_specs=[
            pl.BlockSpec((2, 2, 1024), lambda i: (0, 0, i)),
            pl.BlockSpec(memory_space=pltpu.SMEM),
        ],
        out_specs=pl.BlockSpec((1, 8, 128), lambda i: (i, 0, 0)),
        out_shape=jax.ShapeDtypeStruct((NB, 8, 128), jnp.float32),
    )(o2, b2)


def _sc_edge_body(srcp, dstp, h4f, lsh, ldh, amaxh,
                  acc_out, den_out, denstage,
                  src_sl, dst_sl, tmp_idx, tmp_d, rows, lsbuf, ldbuf,
                  den_h0, den_h1, dtmp, dacc, amv, accs):
    cid = lax.axis_index("c")
    sid = lax.axis_index("s")
    i32 = jnp.int32
    f32 = jnp.float32
    iota16 = lax.iota(i32, 16)
    zf16 = jnp.zeros((16,), f32)

    pltpu.sync_copy(amaxh, amv)

    for c in range(2):
        @pl.when(cid == c)
        def _():
            for p in range(2):
                cb = 2 * c + p
                h0 = 2 * cb
                h1 = h0 + 1
                cb_off = cb * NP
                for half in range(2):
                    lo = half * NPH

                    def _zr(e, c_):
                        for q in range(8):
                            rows[e, pl.ds(q * 16, 16)] = zf16
                        return c_
                    lax.fori_loop(0, B, _zr, 0)
                    for i in range(3):
                        pltpu.sync_copy(
                            rows,
                            accs.at[pl.ds(sid * (NPH // NT) + i * B, B)])

                    def _zd(t, c_):
                        sl = pl.ds(t * 16, 16)
                        den_h0[sl] = zf16
                        den_h1[sl] = zf16
                        return c_
                    lax.fori_loop(0, (NPH + 32) // 16, _zd, 0)
                    plsc.subcore_barrier()
                    amvec = amv[pl.ds(0, 16)]

                    def _slab(s6, c_):
                        pltpu.sync_copy(
                            srcp.at[sid, pl.ds(s6 * CS, CS)], src_sl)
                        pltpu.sync_copy(
                            dstp.at[sid, pl.ds(s6 * CS, CS)], dst_sl)

                        def _chunk(j, cc_):
                            def _off(k, c3_):
                                sl = pl.ds(k * 16, 16)
                                d16 = dst_sl[j, sl]
                                inh = (d16 >= lo) & (d16 < lo + NPH)
                                tmp_d[sl] = jnp.where(
                                    inh, d16 - lo, NPH + k * 16 + iota16)
                                tmp_idx[sl] = src_sl[j, sl] + cb_off
                                return c3_
                            lax.fori_loop(0, 8, _off, 0)
                            pltpu.sync_copy(h4f.at[tmp_idx], rows)
                            pltpu.sync_copy(lsh.at[src_sl.at[j]], lsbuf)
                            pltpu.sync_copy(ldh.at[dst_sl.at[j]], ldbuf)

                            def _inner(k, c3_):
                                dcur = tmp_d[pl.ds(k * 16, 16)]
                                for i in range(16):
                                    e = k * 16 + i
                                    srow = lsbuf[e, pl.ds(0, 16)]
                                    drow = ldbuf[e, pl.ds(0, 16)]
                                    al = srow + drow
                                    al = jnp.where(al >= 0, al, 0.2 * al)
                                    mv = drow + amvec
                                    mv = jnp.where(mv >= 0, mv, 0.2 * mv)
                                    exv = jnp.exp(al - mv)
                                    m0 = exv[h0]
                                    m1 = exv[h1]
                                    for q in range(8):
                                        sl = pl.ds(q * 16, 16)
                                        mm = m0 if q < 4 else m1
                                        rows[e, sl] = rows[e, sl] * mm
                                    nl = dcur[i]
                                    dv0 = den_h0[pl.ds(nl, 16)]
                                    den_h0[pl.ds(nl, 16)] = dv0 + jnp.where(
                                        iota16 == 0, m0, 0.0)
                                    dv1 = den_h1[pl.ds(nl, 16)]
                                    den_h1[pl.ds(nl, 16)] = dv1 + jnp.where(
                                        iota16 == 0, m1, 0.0)
                                return c3_
                            lax.fori_loop(0, 8, _inner, 0)
                            pltpu.sync_copy(rows, accs.at[tmp_d], add=True)
                            return cc_
                        lax.fori_loop(0, CS, _chunk, 0)
                        return c_
                    lax.fori_loop(0, CH // CS, _slab, 0)
                    so0 = ((cb * 2 + 0) * NT + sid) * NP + lo
                    so1 = ((cb * 2 + 1) * NT + sid) * NP + lo
                    pltpu.sync_copy(den_h0.at[pl.ds(0, NPH)],
                                    denstage.at[pl.ds(so0, NPH)])
                    pltpu.sync_copy(den_h1.at[pl.ds(0, NPH)],
                                    denstage.at[pl.ds(so1, NPH)])
                    plsc.subcore_barrier()
                    st = pl.ds(sid * (NPH // NT), NPH // NT)
                    gst = pl.ds(lo + sid * (NPH // NT), NPH // NT)
                    pltpu.sync_copy(accs.at[st], acc_out.at[cb, gst])

                noff = sid * (NP // NT)
                for hh in range(2):
                    def _za(t, c_):
                        dacc[pl.ds(t * 16, 16)] = zf16
                        return c_
                    lax.fori_loop(0, (NP // NT) // 16, _za, 0)
                    for t in range(NT):
                        pltpu.sync_copy(
                            denstage.at[pl.ds(
                                ((cb * 2 + hh) * NT + t) * NP + noff,
                                NP // NT)], dtmp)

                        def _ad(q, c_):
                            sl = pl.ds(q * 16, 16)
                            dacc[sl] = dacc[sl] + dtmp[sl]
                            return c_
                        lax.fori_loop(0, (NP // NT) // 16, _ad, 0)
                    pltpu.sync_copy(
                        dacc,
                        den_out.at[pl.ds((cb * 2 + hh) * NP + noff,
                                         NP // NT)])


def _sc_edge(srcp, dstp, h4f, lsh, ldh, amaxh):
    mesh = plsc.VectorSubcoreMesh(core_axis_name="c", subcore_axis_name="s")
    kern = pl.kernel(
        _sc_edge_body,
        out_type=(jax.ShapeDtypeStruct((4, NP, 128), jnp.float32),
                  jax.ShapeDtypeStruct((4 * 2 * NP,), jnp.float32),
                  jax.ShapeDtypeStruct((4 * 2 * NT * NP,), jnp.float32)),
        mesh=mesh,
        scratch_types=[
            pltpu.VMEM((CS, B), jnp.int32),
            pltpu.VMEM((CS, B), jnp.int32),
            pltpu.VMEM((B,), jnp.int32),
            pltpu.VMEM((B,), jnp.int32),
            pltpu.VMEM((B, 128), jnp.float32),
            pltpu.VMEM((B, 128), jnp.float32),
            pltpu.VMEM((B, 128), jnp.float32),
            pltpu.VMEM((NPH + 144,), jnp.float32),
            pltpu.VMEM((NPH + 144,), jnp.float32),
            pltpu.VMEM((NP // NT,), jnp.float32),
            pltpu.VMEM((NP // NT,), jnp.float32),
            pltpu.VMEM((128,), jnp.float32),
            pltpu.VMEM_SHARED((NPH + 128, 128), jnp.float32),
        ],
    )
    acc4, den4, _ = kern(srcp, dstp, h4f, lsh, ldh, amaxh)
    return acc4, den4


def _sc_edge2_body(srcp, dstp, lth, amaxh,
                   o2_out, o2stage,
                   src_idx, dst_idx, ltS, ltD, pnum, pden, dtmp, dacc, amv):
    cid = lax.axis_index("c")
    sid = lax.axis_index("s")
    i32 = jnp.int32
    f32 = jnp.float32
    iota16 = lax.iota(i32, 16)
    zf16 = jnp.zeros((16,), f32)
    wid = sid * 2 + cid

    pltpu.sync_copy(srcp.at[wid], src_idx)
    pltpu.sync_copy(dstp.at[wid], dst_idx)
    pltpu.sync_copy(amaxh, amv)

    def _zd(t, c_):
        sl = pl.ds(t * 16, 16)
        pnum[sl] = zf16
        pden[sl] = zf16
        return c_
    lax.fori_loop(0, (NP + 16) // 16, _zd, 0)
    amvec = amv[pl.ds(0, 16)]
    am = amvec[0]

    def _chunk(j, c_):
        pltpu.sync_copy(lth.at[src_idx.at[j]], ltS)
        pltpu.sync_copy(lth.at[dst_idx.at[j]], ltD)

        def _inner(k, cc_):
            dcur = dst_idx[j, pl.ds(k * 16, 16)]
            for i in range(16):
                e = k * 16 + i
                srow = ltS[e, pl.ds(0, 16)]
                drow = ltD[e, pl.ds(0, 16)]
                h2s = srow[0]
                as_ = srow[1]
                ad_ = drow[2]
                al = as_ + ad_
                al = jnp.where(al >= 0, al, 0.2 * al)
                mm = ad_ + am
                mm = jnp.where(mm >= 0, mm, 0.2 * mm)
                exv = jnp.exp(al - mm + zf16)
                n = dcur[i]
                nv = pnum[pl.ds(n, 16)]
                pnum[pl.ds(n, 16)] = nv + jnp.where(
                    iota16 == 0, exv * h2s, 0.0)
                dv = pden[pl.ds(n, 16)]
                pden[pl.ds(n, 16)] = dv + jnp.where(iota16 == 0, exv, 0.0)
            return cc_
        lax.fori_loop(0, 8, _inner, 0)
        return c_
    lax.fori_loop(0, CH // 2, _chunk, 0)
    pltpu.sync_copy(pnum.at[pl.ds(0, NP)],
                    o2stage.at[pl.ds(wid * NP, NP)])
    pltpu.sync_copy(pden.at[pl.ds(0, NP)],
                    o2stage.at[pl.ds((32 + wid) * NP, NP)])
    plsc.subcore_barrier()
    noff = sid * (NP // NT)
    for hh in range(2):
        def _za(t, c_):
            dacc[pl.ds(t * 16, 16)] = zf16
            return c_
        lax.fori_loop(0, (NP // NT) // 16, _za, 0)
        for t in range(NT):
            pltpu.sync_copy(
                o2stage.at[pl.ds((hh * 32 + t * 2 + cid) * NP + noff,
                                 NP // NT)], dtmp)

            def _ad(q, c_):
                sl = pl.ds(q * 16, 16)
                dacc[sl] = dacc[sl] + dtmp[sl]
                return c_
            lax.fori_loop(0, (NP // NT) // 16, _ad, 0)
        pltpu.sync_copy(
            dacc, o2_out.at[pl.ds((cid * 2 + hh) * NP + noff, NP // NT)])


def _sc_edge2(srcp32, dstp32, lt, amaxh):
    mesh = plsc.VectorSubcoreMesh(core_axis_name="c", subcore_axis_name="s")
    kern = pl.kernel(
        _sc_edge2_body,
        out_type=(jax.ShapeDtypeStruct((2 * 2 * NP,), jnp.float32),
                  jax.ShapeDtypeStruct((2 * 32 * NP,), jnp.float32)),
        mesh=mesh,
        scratch_types=[
            pltpu.VMEM((CH // 2, B), jnp.int32),
            pltpu.VMEM((CH // 2, B), jnp.int32),
            pltpu.VMEM((B, 128), jnp.float32),
            pltpu.VMEM((B, 128), jnp.float32),
            pltpu.VMEM((NP + 144,), jnp.float32),
            pltpu.VMEM((NP + 144,), jnp.float32),
            pltpu.VMEM((NP // NT,), jnp.float32),
            pltpu.VMEM((NP // NT,), jnp.float32),
            pltpu.VMEM((128,), jnp.float32),
        ],
    )
    o2, _ = kern(srcp32, dstp32, lt, amaxh)
    return o2


def kernel(x, edge_index, W0, as0, ad0, b0, W1, as1, ad1, b1, W2, as2, ad2, b2):
    xp = jnp.pad(x, ((0, NP - N), (0, 0)))
    loop = jnp.arange(N, dtype=jnp.int32)
    npad = EP - E_REAL
    padv = N + (jnp.arange(npad, dtype=jnp.int32) % (NP - N))
    srcp = jnp.concatenate([edge_index[0], loop, padv]).reshape(NT, CH, B)
    dstp = jnp.concatenate([edge_index[1], loop, padv]).reshape(NT, CH, B)
    srcp32 = srcp.reshape(32, CH // 2, B)
    dstp32 = dstp.reshape(32, CH // 2, B)

    h4, ls0, ld0, amax = _pre(xp, W0, as0, ad0)
    amaxh = jnp.pad(amax[:, 0], (0, 120))
    acc4, den4 = _sc_edge(srcp, dstp, h4.reshape(4 * NP, 128),
                          ls0, ld0, amaxh)
    den4 = den4.reshape(4, 2, NP)

    h4b, ls1, ld1, amaxb = _mid(acc4, den4, b0.reshape(1, 512), W1, as1, ad1)
    amaxbh = jnp.pad(amaxb[:, 0], (0, 120))
    acc4b, den4b = _sc_edge(srcp, dstp, h4b.reshape(4 * NP, 128),
                            ls1, ld1, amaxbh)
    den4b = den4b.reshape(4, 2, NP)

    W2p = jnp.pad(W2, ((0, 0), (0, 127)))
    lt, amax2 = _mid2(acc4b, den4b, b1.reshape(1, 512), W2p, as2, ad2)
    amax2h = jnp.pad(amax2[:1, 0], (0, 127))
    o2 = _sc_edge2(srcp32, dstp32, lt, amax2h).reshape(2, 2, NP)
    outp = _fin(o2, b2.reshape(1, 1))
    return outp.reshape(NP)[:N]


# speedup vs baseline: 10.9192x; 1.3391x over previous
import jax
import jax.numpy as jnp
from jax import lax
from jax.experimental import pallas as pl
from jax.experimental.pallas import tpu as pltpu
from jax.experimental.pallas import tpu_sc as plsc

N = 10000
NP = 10240
NPH = NP // 2
NT = 16
E_REAL = 330000
CH = 168
CS = 24
B = 128
EP = NT * CH * B
NB = NP // 1024
CAP = 8 * CS * B
NEG = -1e30


def _attn_tail(h, as_w, ad_w, i, h4_ref, ls_ref, ld_ref, amax_ref):
    hr = h.reshape(1024, 8, 64)
    a_s = jnp.sum(hr * as_w[None], axis=-1)
    a_d = jnp.sum(hr * ad_w[None], axis=-1)
    rid = lax.broadcasted_iota(jnp.int32, (1024, 8), 0) + i * 1024
    a_s = jnp.where(rid >= N, NEG, a_s)
    h4_ref[...] = h.reshape(1024, 4, 128).transpose(1, 0, 2)
    ls_ref[...] = jnp.pad(a_s, ((0, 0), (0, 120)))
    ld_ref[...] = jnp.pad(a_d, ((0, 0), (0, 120)))

    @pl.when(i == 0)
    def _():
        amax_ref[...] = jnp.full((8, 128), NEG, jnp.float32)

    bm = jnp.max(a_s, axis=0)
    amax_ref[...] = jnp.maximum(amax_ref[...], bm[:, None])


def _pre_kernel(x_ref, w_ref, as_ref, ad_ref,
                h4_ref, ls_ref, ld_ref, amax_ref):
    i = pl.program_id(0)
    h = jnp.dot(x_ref[...], w_ref[...], preferred_element_type=jnp.float32)
    _attn_tail(h, as_ref[...], ad_ref[...], i,
               h4_ref, ls_ref, ld_ref, amax_ref)


def _finalize_prev(acc_ref, den_ref, b_ref, i):
    acc = acc_ref[...]
    den = den_ref[...]
    a = acc.transpose(1, 0, 2).reshape(1024, 512)
    dn = den.reshape(8, 1024).T
    hsel = (lax.broadcasted_iota(jnp.int32, (8, 512), 1) // 64 ==
            lax.broadcasted_iota(jnp.int32, (8, 512), 0))
    db = jnp.dot(dn, hsel.astype(jnp.float32),
                 preferred_element_type=jnp.float32)
    y = a / (db + 1e-16) + b_ref[...]
    y = jnp.where(y > 0, y, jnp.exp(jnp.minimum(y, 0.0)) - 1.0)
    rid = lax.broadcasted_iota(jnp.int32, (1024, 512), 0) + i * 1024
    return jnp.where(rid >= N, 0.0, y)


def _mid_kernel(acc_ref, den_ref, b_ref, w_ref, as_ref, ad_ref,
                h4_ref, ls_ref, ld_ref, amax_ref):
    i = pl.program_id(0)
    y = _finalize_prev(acc_ref, den_ref, b_ref, i)
    h = jnp.dot(y, w_ref[...], preferred_element_type=jnp.float32)
    _attn_tail(h, as_ref[...], ad_ref[...], i,
               h4_ref, ls_ref, ld_ref, amax_ref)


def _mid2_kernel(acc_ref, den_ref, b_ref, w_ref, as_ref, ad_ref,
                 lt_ref, amax_ref):
    i = pl.program_id(0)
    y = _finalize_prev(acc_ref, den_ref, b_ref, i)
    h2f = jnp.dot(y, w_ref[...], preferred_element_type=jnp.float32)
    h2 = h2f[:, 0:1]
    asv = h2 * as_ref[0, 0]
    adv = h2 * ad_ref[0, 0]
    rid = lax.broadcasted_iota(jnp.int32, (1024, 1), 0) + i * 1024
    asv = jnp.where(rid >= N, NEG, asv)
    lt_ref[...] = jnp.pad(jnp.concatenate([h2, asv, adv], axis=1),
                          ((0, 0), (0, 125)))

    @pl.when(i == 0)
    def _():
        amax_ref[...] = jnp.full((8, 128), NEG, jnp.float32)

    amax_ref[...] = jnp.maximum(amax_ref[...], jnp.max(asv))


def _fin_kernel(o2_ref, b2_ref, out_ref):
    o = o2_ref[...]
    num = o[0, 0] + o[1, 0]
    den = o[0, 1] + o[1, 1]
    r = num / (den + 1e-16) + b2_ref[0, 0]
    out_ref[...] = jax.nn.sigmoid(r).reshape(1, 8, 128)


def _pre(xp, W0, as0, ad0):
    return pl.pallas_call(
        _pre_kernel,
        grid=(NB,),
        in_specs=[
            pl.BlockSpec((1024, 128), lambda i: (i, 0)),
            pl.BlockSpec((128, 512), lambda i: (0, 0)),
            pl.BlockSpec((8, 64), lambda i: (0, 0)),
            pl.BlockSpec((8, 64), lambda i: (0, 0)),
        ],
        out_specs=[
            pl.BlockSpec((4, 1024, 128), lambda i: (0, i, 0)),
            pl.BlockSpec((1024, 128), lambda i: (i, 0)),
            pl.BlockSpec((1024, 128), lambda i: (i, 0)),
            pl.BlockSpec((8, 128), lambda i: (0, 0)),
        ],
        out_shape=[
            jax.ShapeDtypeStruct((4, NP, 128), jnp.float32),
            jax.ShapeDtypeStruct((NP, 128), jnp.float32),
            jax.ShapeDtypeStruct((NP, 128), jnp.float32),
            jax.ShapeDtypeStruct((8, 128), jnp.float32),
        ],
    )(xp, W0, as0, ad0)


def _mid(acc4, den4, b, W, as_w, ad_w):
    return pl.pallas_call(
        _mid_kernel,
        grid=(NB,),
        in_specs=[
            pl.BlockSpec((4, 1024, 128), lambda i: (0, i, 0)),
            pl.BlockSpec((4, 2, 1024), lambda i: (0, 0, i)),
            pl.BlockSpec((1, 512), lambda i: (0, 0)),
            pl.BlockSpec((512, 512), lambda i: (0, 0)),
            pl.BlockSpec((8, 64), lambda i: (0, 0)),
            pl.BlockSpec((8, 64), lambda i: (0, 0)),
        ],
        out_specs=[
            pl.BlockSpec((4, 1024, 128), lambda i: (0, i, 0)),
            pl.BlockSpec((1024, 128), lambda i: (i, 0)),
            pl.BlockSpec((1024, 128), lambda i: (i, 0)),
            pl.BlockSpec((8, 128), lambda i: (0, 0)),
        ],
        out_shape=[
            jax.ShapeDtypeStruct((4, NP, 128), jnp.float32),
            jax.ShapeDtypeStruct((NP, 128), jnp.float32),
            jax.ShapeDtypeStruct((NP, 128), jnp.float32),
            jax.ShapeDtypeStruct((8, 128), jnp.float32),
        ],
    )(acc4, den4, b, W, as_w, ad_w)


def _mid2(acc4, den4, b, W2p, as2, ad2):
    return pl.pallas_call(
        _mid2_kernel,
        grid=(NB,),
        in_specs=[
            pl.BlockSpec((4, 1024, 128), lambda i: (0, i, 0)),
            pl.BlockSpec((4, 2, 1024), lambda i: (0, 0, i)),
            pl.BlockSpec((1, 512), lambda i: (0, 0)),
            pl.BlockSpec((512, 128), lambda i: (0, 0)),
            pl.BlockSpec(memory_space=pltpu.SMEM),
            pl.BlockSpec(memory_space=pltpu.SMEM),
        ],
        out_specs=[
            pl.BlockSpec((1024, 128), lambda i: (i, 0)),
            pl.BlockSpec((8, 128), lambda i: (0, 0)),
        ],
        out_shape=[
            jax.ShapeDtypeStruct((NP, 128), jnp.float32),
            jax.ShapeDtypeStruct((8, 128), jnp.float32),
        ],
    )(acc4, den4, b, W2p, as2, ad2)


def _fin(o2, b2):
    return pl.pallas_call(
        _fin_kernel,
        grid=(NB,),
        in_specs=[
            pl.BlockSpec((2, 2, 1024), lambda i: (0, 0, i)),
            pl.BlockSpec(memory_space=pltpu.SMEM),
        ],
        out_specs=pl.BlockSpec((1, 8, 128), lambda i: (i, 0, 0)),
        out_shape=jax.ShapeDtypeStruct((NB, 8, 128), jnp.float32),
    )(o2, b2)


def _sc_prep_body(srcp, dstp, plists, pcnt,
                  src_sl, dst_sl, lbuf, cv):
    cid = lax.axis_index("c")
    sid = lax.axis_index("s")
    i32 = jnp.int32
    iota16 = lax.iota(i32, 16)
    lo = cid * NPH

    def _fill(t, c_):
        dl = lo + ((t * 16 + iota16) & 2047)
        lbuf[pl.ds(t * 16, 16)] = N + dl * 16384
        return c_
    lax.fori_loop(0, CAP // 16, _fill, 0)

    def _slab(s6, p):
        pltpu.sync_copy(srcp.at[sid, pl.ds(s6 * CS, CS)], src_sl)
        pltpu.sync_copy(dstp.at[sid, pl.ds(s6 * CS, CS)], dst_sl)

        def _chunk(j, p):
            def _grp(k, p):
                sl = pl.ds(k * 16, 16)
                scur = src_sl[j, sl]
                dcur = dst_sl[j, sl]
                vcur = scur + dcur * 16384
                for i in range(16):
                    v = vcur[i]
                    d = dcur[i]
                    win = lbuf[pl.ds(p, 16)]
                    lbuf[pl.ds(p, 16)] = jnp.where(iota16 == 0, v, win)
                    mh = jnp.where(d >= NPH, 1, 0)
                    p = p + jnp.where(mh == cid, 1, 0)
                return p
            return lax.fori_loop(0, 8, _grp, p)
        return lax.fori_loop(0, CS, _chunk, p)
    p = lax.fori_loop(0, CH // CS, _slab, 0)
    dl = lo + (iota16 & 2047)
    lbuf[pl.ds(p, 16)] = N + dl * 16384

    off = (cid * NT + sid) * CAP
    pltpu.sync_copy(lbuf.at[pl.ds(0, CAP)], plists.at[pl.ds(off, CAP)])

    def _cw(t, c_):
        cv[pl.ds(t * 16, 16)] = jnp.zeros((16,), i32)
        return c_
    lax.fori_loop(0, 8, _cw, 0)
    cv[pl.ds(0, 16)] = jnp.where(iota16 == 0, p, 0)
    pltpu.sync_copy(cv, pcnt.at[pl.ds((cid * NT + sid) * 128, 128)])


def _sc_prep(srcp, dstp):
    mesh = plsc.VectorSubcoreMesh(core_axis_name="c", subcore_axis_name="s")
    kern = pl.kernel(
        _sc_prep_body,
        out_type=(jax.ShapeDtypeStruct((2 * NT * CAP,), jnp.int32),
                  jax.ShapeDtypeStruct((2 * NT * 128,), jnp.int32)),
        mesh=mesh,
        scratch_types=[
            pltpu.VMEM((CS, B), jnp.int32),
            pltpu.VMEM((CS, B), jnp.int32),
            pltpu.VMEM((CAP + 16,), jnp.int32),
            pltpu.VMEM((128,), jnp.int32),
        ],
    )
    return kern(srcp, dstp)


def _sc_edge_body(plists, pcnt, h4f, lsh, ldh, amaxh,
                  acc_out, den_out, denstage,
                  slab_v, cv, tmp_s, tmp_dr, tmp_idx, tmp_dl,
                  rows, lsbuf, ldbuf,
                  den_h0, den_h1, dtmp, dacc, amv, accs):
    cid = lax.axis_index("c")
    sid = lax.axis_index("s")
    i32 = jnp.int32
    f32 = jnp.float32
    iota16 = lax.iota(i32, 16)
    zf16 = jnp.zeros((16,), f32)

    pltpu.sync_copy(amaxh, amv)

    for c in range(2):
        @pl.when(cid == c)
        def _():
            for p in range(2):
                cb = 2 * c + p
                h0 = 2 * cb
                h1 = h0 + 1
                cb_off = cb * NP
                for half in range(2):
                    lo = half * NPH
                    pltpu.sync_copy(
                        pcnt.at[pl.ds((half * NT + sid) * 128, 128)], cv)
                    cnt = cv[pl.ds(0, 16)][0]
                    nch = lax.shift_right_logical(cnt + 127, 7)
                    list_off = (half * NT + sid) * CAP

                    def _zr(e, c_):
                        for q in range(8):
                            rows[e, pl.ds(q * 16, 16)] = zf16
                        return c_
                    lax.fori_loop(0, B, _zr, 0)
                    for i in range(3):
                        pltpu.sync_copy(
                            rows,
                            accs.at[pl.ds(sid * (NPH // NT) + i * B, B)])

                    def _zd(t, c_):
                        sl = pl.ds(t * 16, 16)
                        den_h0[sl] = zf16
                        den_h1[sl] = zf16
                        return c_
                    lax.fori_loop(0, (NPH + 32) // 16, _zd, 0)
                    plsc.subcore_barrier()
                    amvec = amv[pl.ds(0, 16)]

                    def _slab(s6, c_):
                        pltpu.sync_copy(
                            plists.at[pl.ds(list_off + s6 * (CS * B),
                                            CS * B)], slab_v)
                        jmax = jnp.clip(nch - s6 * CS, 0, CS)

                        def _chunk(j, cc_):
                            def _off(k, c3_):
                                sl = pl.ds(k * 16, 16)
                                v16 = slab_v[pl.ds(j * B + k * 16, 16)]
                                d16 = lax.shift_right_logical(v16, 14)
                                s16 = v16 & 16383
                                tmp_s[sl] = s16
                                tmp_dr[sl] = d16
                                tmp_idx[sl] = s16 + cb_off
                                tmp_dl[sl] = d16 - lo
                                return c3_
                            lax.fori_loop(0, 8, _off, 0)
                            pltpu.sync_copy(h4f.at[tmp_idx], rows)
                            pltpu.sync_copy(lsh.at[tmp_s], lsbuf)
                            pltpu.sync_copy(ldh.at[tmp_dr], ldbuf)

                            def _inner(k, c3_):
                                dcur = tmp_dl[pl.ds(k * 16, 16)]
                                for i in range(16):
                                    e = k * 16 + i
                                    srow = lsbuf[e, pl.ds(0, 16)]
                                    drow = ldbuf[e, pl.ds(0, 16)]
                                    al = srow + drow
                                    al = jnp.where(al >= 0, al, 0.2 * al)
                                    mv = drow + amvec
                                    mv = jnp.where(mv >= 0, mv, 0.2 * mv)
                                    exv = jnp.exp(al - mv)
                                    m0 = exv[h0]
                                    m1 = exv[h1]
                                    for q in range(8):
                                        sl = pl.ds(q * 16, 16)
                                        mm = m0 if q < 4 else m1
                                        rows[e, sl] = rows[e, sl] * mm
                                    nl = dcur[i]
                                    dv0 = den_h0[pl.ds(nl, 16)]
                                    den_h0[pl.ds(nl, 16)] = dv0 + jnp.where(
                                        iota16 == 0, m0, 0.0)
                                    dv1 = den_h1[pl.ds(nl, 16)]
                                    den_h1[pl.ds(nl, 16)] = dv1 + jnp.where(
                                        iota16 == 0, m1, 0.0)
                                return c3_
                            lax.fori_loop(0, 8, _inner, 0)
                            pltpu.sync_copy(rows, accs.at[tmp_dl], add=True)
                            return cc_
                        lax.fori_loop(0, jmax, _chunk, 0)
                        return c_
                    lax.fori_loop(0, 8, _slab, 0)
                    so0 = ((cb * 2 + 0) * NT + sid) * NP + lo
                    so1 = ((cb * 2 + 1) * NT + sid) * NP + lo
                    pltpu.sync_copy(den_h0.at[pl.ds(0, NPH)],
                                    denstage.at[pl.ds(so0, NPH)])
                    pltpu.sync_copy(den_h1.at[pl.ds(0, NPH)],
                                    denstage.at[pl.ds(so1, NPH)])
                    plsc.subcore_barrier()
                    st = pl.ds(sid * (NPH // NT), NPH // NT)
                    gst = pl.ds(lo + sid * (NPH // NT), NPH // NT)
                    pltpu.sync_copy(accs.at[st], acc_out.at[cb, gst])

                noff = sid * (NP // NT)
                for hh in range(2):
                    def _za(t, c_):
                        dacc[pl.ds(t * 16, 16)] = zf16
                        return c_
                    lax.fori_loop(0, (NP // NT) // 16, _za, 0)
                    for t in range(NT):
                        pltpu.sync_copy(
                            denstage.at[pl.ds(
                                ((cb * 2 + hh) * NT + t) * NP + noff,
                                NP // NT)], dtmp)

                        def _ad(q, c_):
                            sl = pl.ds(q * 16, 16)
                            dacc[sl] = dacc[sl] + dtmp[sl]
                            return c_
                        lax.fori_loop(0, (NP // NT) // 16, _ad, 0)
                    pltpu.sync_copy(
                        dacc,
                        den_out.at[pl.ds((cb * 2 + hh) * NP + noff,
                                         NP // NT)])


def _sc_edge(plists, pcnt, h4f, lsh, ldh, amaxh):
    mesh = plsc.VectorSubcoreMesh(core_axis_name="c", subcore_axis_name="s")
    kern = pl.kernel(
        _sc_edge_body,
        out_type=(jax.ShapeDtypeStruct((4, NP, 128), jnp.float32),
                  jax.ShapeDtypeStruct((4 * 2 * NP,), jnp.float32),
                  jax.ShapeDtypeStruct((4 * 2 * NT * NP,), jnp.float32)),
        mesh=mesh,
        scratch_types=[
            pltpu.VMEM((CS * B,), jnp.int32),
            pltpu.VMEM((128,), jnp.int32),
            pltpu.VMEM((B,), jnp.int32),
            pltpu.VMEM((B,), jnp.int32),
            pltpu.VMEM((B,), jnp.int32),
            pltpu.VMEM((B,), jnp.int32),
            pltpu.VMEM((B, 128), jnp.float32),
            pltpu.VMEM((B, 128), jnp.float32),
            pltpu.VMEM((B, 128), jnp.float32),
            pltpu.VMEM((NPH + 144,), jnp.float32),
            pltpu.VMEM((NPH + 144,), jnp.float32),
            pltpu.VMEM((NP // NT,), jnp.float32),
            pltpu.VMEM((NP // NT,), jnp.float32),
            pltpu.VMEM((128,), jnp.float32),
            pltpu.VMEM_SHARED((NPH + 128, 128), jnp.float32),
        ],
    )
    acc4, den4, _ = kern(plists, pcnt, h4f, lsh, ldh, amaxh)
    return acc4, den4


def _sc_edge2_body(srcp, dstp, lth, amaxh,
                   o2_out, o2stage,
                   src_idx, dst_idx, ltS, ltD, pnum, pden, dtmp, dacc, amv):
    cid = lax.axis_index("c")
    sid = lax.axis_index("s")
    i32 = jnp.int32
    f32 = jnp.float32
    iota16 = lax.iota(i32, 16)
    zf16 = jnp.zeros((16,), f32)
    wid = sid * 2 + cid

    pltpu.sync_copy(srcp.at[wid], src_idx)
    pltpu.sync_copy(dstp.at[wid], dst_idx)
    pltpu.sync_copy(amaxh, amv)

    def _zd(t, c_):
        sl = pl.ds(t * 16, 16)
        pnum[sl] = zf16
        pden[sl] = zf16
        return c_
    lax.fori_loop(0, (NP + 16) // 16, _zd, 0)
    amvec = amv[pl.ds(0, 16)]
    am = amvec[0]

    def _chunk(j, c_):
        pltpu.sync_copy(lth.at[src_idx.at[j]], ltS)
        pltpu.sync_copy(lth.at[dst_idx.at[j]], ltD)

        def _inner(k, cc_):
            dcur = dst_idx[j, pl.ds(k * 16, 16)]
            for i in range(16):
                e = k * 16 + i
                srow = ltS[e, pl.ds(0, 16)]
                drow = ltD[e, pl.ds(0, 16)]
                h2s = srow[0]
                as_ = srow[1]
                ad_ = drow[2]
                al = as_ + ad_
                al = jnp.where(al >= 0, al, 0.2 * al)
                mm = ad_ + am
                mm = jnp.where(mm >= 0, mm, 0.2 * mm)
                exv = jnp.exp(al - mm + zf16)
                n = dcur[i]
                nv = pnum[pl.ds(n, 16)]
                pnum[pl.ds(n, 16)] = nv + jnp.where(
                    iota16 == 0, exv * h2s, 0.0)
                dv = pden[pl.ds(n, 16)]
                pden[pl.ds(n, 16)] = dv + jnp.where(iota16 == 0, exv, 0.0)
            return cc_
        lax.fori_loop(0, 8, _inner, 0)
        return c_
    lax.fori_loop(0, CH // 2, _chunk, 0)
    pltpu.sync_copy(pnum.at[pl.ds(0, NP)],
                    o2stage.at[pl.ds(wid * NP, NP)])
    pltpu.sync_copy(pden.at[pl.ds(0, NP)],
                    o2stage.at[pl.ds((32 + wid) * NP, NP)])
    plsc.subcore_barrier()
    noff = sid * (NP // NT)
    for hh in range(2):
        def _za(t, c_):
            dacc[pl.ds(t * 16, 16)] = zf16
            return c_
        lax.fori_loop(0, (NP // NT) // 16, _za, 0)
        for t in range(NT):
            pltpu.sync_copy(
                o2stage.at[pl.ds((hh * 32 + t * 2 + cid) * NP + noff,
                                 NP // NT)], dtmp)

            def _ad(q, c_):
                sl = pl.ds(q * 16, 16)
                dacc[sl] = dacc[sl] + dtmp[sl]
                return c_
            lax.fori_loop(0, (NP // NT) // 16, _ad, 0)
        pltpu.sync_copy(
            dacc, o2_out.at[pl.ds((cid * 2 + hh) * NP + noff, NP // NT)])


def _sc_edge2(srcp32, dstp32, lt, amaxh):
    mesh = plsc.VectorSubcoreMesh(core_axis_name="c", subcore_axis_name="s")
    kern = pl.kernel(
        _sc_edge2_body,
        out_type=(jax.ShapeDtypeStruct((2 * 2 * NP,), jnp.float32),
                  jax.ShapeDtypeStruct((2 * 32 * NP,), jnp.float32)),
        mesh=mesh,
        scratch_types=[
            pltpu.VMEM((CH // 2, B), jnp.int32),
            pltpu.VMEM((CH // 2, B), jnp.int32),
            pltpu.VMEM((B, 128), jnp.float32),
            pltpu.VMEM((B, 128), jnp.float32),
            pltpu.VMEM((NP + 144,), jnp.float32),
            pltpu.VMEM((NP + 144,), jnp.float32),
            pltpu.VMEM((NP // NT,), jnp.float32),
            pltpu.VMEM((NP // NT,), jnp.float32),
            pltpu.VMEM((128,), jnp.float32),
        ],
    )
    o2, _ = kern(srcp32, dstp32, lt, amaxh)
    return o2


def kernel(x, edge_index, W0, as0, ad0, b0, W1, as1, ad1, b1, W2, as2, ad2, b2):
    xp = jnp.pad(x, ((0, NP - N), (0, 0)))
    loop = jnp.arange(N, dtype=jnp.int32)
    npad = EP - E_REAL
    padv = N + (jnp.arange(npad, dtype=jnp.int32) % (NP - N))
    srcp = jnp.concatenate([edge_index[0], loop, padv]).reshape(NT, CH, B)
    dstp = jnp.concatenate([edge_index[1], loop, padv]).reshape(NT, CH, B)
    srcp32 = srcp.reshape(32, CH // 2, B)
    dstp32 = dstp.reshape(32, CH // 2, B)

    plists, pcnt = _sc_prep(srcp, dstp)
    h4, ls0, ld0, amax = _pre(xp, W0, as0, ad0)
    amaxh = jnp.pad(amax[:, 0], (0, 120))
    acc4, den4 = _sc_edge(plists, pcnt, h4.reshape(4 * NP, 128),
                          ls0, ld0, amaxh)
    den4 = den4.reshape(4, 2, NP)

    h4b, ls1, ld1, amaxb = _mid(acc4, den4, b0.reshape(1, 512), W1, as1, ad1)
    amaxbh = jnp.pad(amaxb[:, 0], (0, 120))
    acc4b, den4b = _sc_edge(plists, pcnt, h4b.reshape(4 * NP, 128),
                            ls1, ld1, amaxbh)
    den4b = den4b.reshape(4, 2, NP)

    W2p = jnp.pad(W2, ((0, 0), (0, 127)))
    lt, amax2 = _mid2(acc4b, den4b, b1.reshape(1, 512), W2p, as2, ad2)
    amax2h = jnp.pad(amax2[:1, 0], (0, 127))
    o2 = _sc_edge2(srcp32, dstp32, lt, amax2h).reshape(2, 2, NP)
    outp = _fin(o2, b2.reshape(1, 1))
    return outp.reshape(NP)[:N]


# speedup vs baseline: 12.8162x; 1.1737x over previous
import jax
import jax.numpy as jnp
from jax import lax
from jax.experimental import pallas as pl
from jax.experimental.pallas import tpu as pltpu
from jax.experimental.pallas import tpu_sc as plsc

N = 10000
NP = 10240
NPH = NP // 2
NT = 16
E_REAL = 330000
CH = 168
CS = 24
B = 128
EP = NT * CH * B
NB = NP // 1024
CAP = 8 * CS * B
NEG = -1e30


def _attn_tail(h, as_w, ad_w, i, h4_ref, ls_ref, ld_ref, amax_ref):
    hr = h.reshape(1024, 8, 64)
    a_s = jnp.sum(hr * as_w[None], axis=-1)
    a_d = jnp.sum(hr * ad_w[None], axis=-1)
    rid = lax.broadcasted_iota(jnp.int32, (1024, 8), 0) + i * 1024
    a_s = jnp.where(rid >= N, NEG, a_s)
    h4_ref[...] = h.reshape(1024, 4, 128).transpose(1, 0, 2)
    ls_ref[...] = jnp.pad(a_s, ((0, 0), (0, 120)))
    ld_ref[...] = jnp.pad(a_d, ((0, 0), (0, 120)))

    @pl.when(i == 0)
    def _():
        amax_ref[...] = jnp.full((8, 128), NEG, jnp.float32)

    bm = jnp.max(a_s, axis=0)
    amax_ref[...] = jnp.maximum(amax_ref[...], bm[:, None])


def _pre_kernel(x_ref, w_ref, as_ref, ad_ref,
                h4_ref, ls_ref, ld_ref, amax_ref):
    i = pl.program_id(0)
    h = jnp.dot(x_ref[...], w_ref[...], preferred_element_type=jnp.float32)
    _attn_tail(h, as_ref[...], ad_ref[...], i,
               h4_ref, ls_ref, ld_ref, amax_ref)


def _finalize_prev(acc_ref, den_ref, b_ref, i):
    acc = acc_ref[...]
    den = den_ref[...]
    a = acc.transpose(1, 0, 2).reshape(1024, 512)
    dn = den.reshape(8, 1024).T
    hsel = (lax.broadcasted_iota(jnp.int32, (8, 512), 1) // 64 ==
            lax.broadcasted_iota(jnp.int32, (8, 512), 0))
    db = jnp.dot(dn, hsel.astype(jnp.float32),
                 preferred_element_type=jnp.float32)
    y = a / (db + 1e-16) + b_ref[...]
    y = jnp.where(y > 0, y, jnp.exp(jnp.minimum(y, 0.0)) - 1.0)
    rid = lax.broadcasted_iota(jnp.int32, (1024, 512), 0) + i * 1024
    return jnp.where(rid >= N, 0.0, y)


def _mid_kernel(acc_ref, den_ref, b_ref, w_ref, as_ref, ad_ref,
                h4_ref, ls_ref, ld_ref, amax_ref):
    i = pl.program_id(0)
    y = _finalize_prev(acc_ref, den_ref, b_ref, i)
    h = jnp.dot(y, w_ref[...], preferred_element_type=jnp.float32)
    _attn_tail(h, as_ref[...], ad_ref[...], i,
               h4_ref, ls_ref, ld_ref, amax_ref)


def _mid2_kernel(acc_ref, den_ref, b_ref, w_ref, as_ref, ad_ref,
                 lt_ref, amax_ref):
    i = pl.program_id(0)
    y = _finalize_prev(acc_ref, den_ref, b_ref, i)
    h2f = jnp.dot(y, w_ref[...], preferred_element_type=jnp.float32)
    h2 = h2f[:, 0:1]
    asv = h2 * as_ref[0, 0]
    adv = h2 * ad_ref[0, 0]
    rid = lax.broadcasted_iota(jnp.int32, (1024, 1), 0) + i * 1024
    asv = jnp.where(rid >= N, NEG, asv)
    lt_ref[...] = jnp.pad(jnp.concatenate([h2, asv, adv], axis=1),
                          ((0, 0), (0, 125)))

    @pl.when(i == 0)
    def _():
        amax_ref[...] = jnp.full((8, 128), NEG, jnp.float32)

    amax_ref[...] = jnp.maximum(amax_ref[...], jnp.max(asv))


def _fin_kernel(o2_ref, b2_ref, out_ref):
    o = o2_ref[...]
    num = o[0, 0] + o[1, 0]
    den = o[0, 1] + o[1, 1]
    r = num / (den + 1e-16) + b2_ref[0, 0]
    out_ref[...] = jax.nn.sigmoid(r).reshape(1, 8, 128)


def _pre(xp, W0, as0, ad0):
    return pl.pallas_call(
        _pre_kernel,
        grid=(NB,),
        in_specs=[
            pl.BlockSpec((1024, 128), lambda i: (i, 0)),
            pl.BlockSpec((128, 512), lambda i: (0, 0)),
            pl.BlockSpec((8, 64), lambda i: (0, 0)),
            pl.BlockSpec((8, 64), lambda i: (0, 0)),
        ],
        out_specs=[
            pl.BlockSpec((4, 1024, 128), lambda i: (0, i, 0)),
            pl.BlockSpec((1024, 128), lambda i: (i, 0)),
            pl.BlockSpec((1024, 128), lambda i: (i, 0)),
            pl.BlockSpec((8, 128), lambda i: (0, 0)),
        ],
        out_shape=[
            jax.ShapeDtypeStruct((4, NP, 128), jnp.float32),
            jax.ShapeDtypeStruct((NP, 128), jnp.float32),
            jax.ShapeDtypeStruct((NP, 128), jnp.float32),
            jax.ShapeDtypeStruct((8, 128), jnp.float32),
        ],
    )(xp, W0, as0, ad0)


def _mid(acc4, den4, b, W, as_w, ad_w):
    return pl.pallas_call(
        _mid_kernel,
        grid=(NB,),
        in_specs=[
            pl.BlockSpec((4, 1024, 128), lambda i: (0, i, 0)),
            pl.BlockSpec((4, 2, 1024), lambda i: (0, 0, i)),
            pl.BlockSpec((1, 512), lambda i: (0, 0)),
            pl.BlockSpec((512, 512), lambda i: (0, 0)),
            pl.BlockSpec((8, 64), lambda i: (0, 0)),
            pl.BlockSpec((8, 64), lambda i: (0, 0)),
        ],
        out_specs=[
            pl.BlockSpec((4, 1024, 128), lambda i: (0, i, 0)),
            pl.BlockSpec((1024, 128), lambda i: (i, 0)),
            pl.BlockSpec((1024, 128), lambda i: (i, 0)),
            pl.BlockSpec((8, 128), lambda i: (0, 0)),
        ],
        out_shape=[
            jax.ShapeDtypeStruct((4, NP, 128), jnp.float32),
            jax.ShapeDtypeStruct((NP, 128), jnp.float32),
            jax.ShapeDtypeStruct((NP, 128), jnp.float32),
            jax.ShapeDtypeStruct((8, 128), jnp.float32),
        ],
    )(acc4, den4, b, W, as_w, ad_w)


def _mid2(acc4, den4, b, W2p, as2, ad2):
    return pl.pallas_call(
        _mid2_kernel,
        grid=(NB,),
        in_specs=[
            pl.BlockSpec((4, 1024, 128), lambda i: (0, i, 0)),
            pl.BlockSpec((4, 2, 1024), lambda i: (0, 0, i)),
            pl.BlockSpec((1, 512), lambda i: (0, 0)),
            pl.BlockSpec((512, 128), lambda i: (0, 0)),
            pl.BlockSpec(memory_space=pltpu.SMEM),
            pl.BlockSpec(memory_space=pltpu.SMEM),
        ],
        out_specs=[
            pl.BlockSpec((1024, 128), lambda i: (i, 0)),
            pl.BlockSpec((8, 128), lambda i: (0, 0)),
        ],
        out_shape=[
            jax.ShapeDtypeStruct((NP, 128), jnp.float32),
            jax.ShapeDtypeStruct((8, 128), jnp.float32),
        ],
    )(acc4, den4, b, W2p, as2, ad2)


def _fin(o2, b2):
    return pl.pallas_call(
        _fin_kernel,
        grid=(NB,),
        in_specs=[
            pl.BlockSpec((2, 2, 1024), lambda i: (0, 0, i)),
            pl.BlockSpec(memory_space=pltpu.SMEM),
        ],
        out_specs=pl.BlockSpec((1, 8, 128), lambda i: (i, 0, 0)),
        out_shape=jax.ShapeDtypeStruct((NB, 8, 128), jnp.float32),
    )(o2, b2)


def _sc_prep_body(srcp, dstp, plists, pcnt,
                  src_sl, dst_sl, lbuf, cv):
    cid = lax.axis_index("c")
    sid = lax.axis_index("s")
    i32 = jnp.int32
    iota16 = lax.iota(i32, 16)
    lo = cid * NPH

    def _fill(t, c_):
        dl = lo + ((t * 16 + iota16) & 2047)
        lbuf[pl.ds(t * 16, 16)] = N + dl * 16384
        return c_
    lax.fori_loop(0, CAP // 16, _fill, 0)

    def _slab(s6, p):
        pltpu.sync_copy(srcp.at[sid, pl.ds(s6 * CS, CS)], src_sl)
        pltpu.sync_copy(dstp.at[sid, pl.ds(s6 * CS, CS)], dst_sl)

        def _chunk(j, p):
            def _grp(k, p):
                sl = pl.ds(k * 16, 16)
                scur = src_sl[j, sl]
                dcur = dst_sl[j, sl]
                vcur = scur + dcur * 16384
                for i in range(16):
                    v = vcur[i]
                    d = dcur[i]
                    win = lbuf[pl.ds(p, 16)]
                    lbuf[pl.ds(p, 16)] = jnp.where(iota16 == 0, v, win)
                    mh = jnp.where(d >= NPH, 1, 0)
                    p = p + jnp.where(mh == cid, 1, 0)
                return p
            return lax.fori_loop(0, 8, _grp, p)
        return lax.fori_loop(0, CS, _chunk, p)
    p = lax.fori_loop(0, CH // CS, _slab, 0)
    dl = lo + (iota16 & 2047)
    lbuf[pl.ds(p, 16)] = N + dl * 16384

    off = (cid * NT + sid) * CAP
    pltpu.sync_copy(lbuf.at[pl.ds(0, CAP)], plists.at[pl.ds(off, CAP)])

    def _cw(t, c_):
        cv[pl.ds(t * 16, 16)] = jnp.zeros((16,), i32)
        return c_
    lax.fori_loop(0, 8, _cw, 0)
    cv[pl.ds(0, 16)] = jnp.where(iota16 == 0, p, 0)
    pltpu.sync_copy(cv, pcnt.at[pl.ds((cid * NT + sid) * 128, 128)])


def _sc_prep(srcp, dstp):
    mesh = plsc.VectorSubcoreMesh(core_axis_name="c", subcore_axis_name="s")
    kern = pl.kernel(
        _sc_prep_body,
        out_type=(jax.ShapeDtypeStruct((2 * NT * CAP,), jnp.int32),
                  jax.ShapeDtypeStruct((2 * NT * 128,), jnp.int32)),
        mesh=mesh,
        scratch_types=[
            pltpu.VMEM((CS, B), jnp.int32),
            pltpu.VMEM((CS, B), jnp.int32),
            pltpu.VMEM((CAP + 16,), jnp.int32),
            pltpu.VMEM((128,), jnp.int32),
        ],
    )
    return kern(srcp, dstp)


def _sc_edge_body(plists, pcnt, h4f, lsh, ldh, amaxh,
                  acc_out, den_out, denstage,
                  slab_v, cv, tmp_s, tmp_dr, tmp_idx, tmp_dl,
                  rows, lsbuf, ldbuf,
                  den_h0, den_h1, dtmp, dacc, amv, sem1, sem2, sem3, accs):
    cid = lax.axis_index("c")
    sid = lax.axis_index("s")
    i32 = jnp.int32
    f32 = jnp.float32
    iota16 = lax.iota(i32, 16)
    zf16 = jnp.zeros((16,), f32)

    pltpu.sync_copy(amaxh, amv)

    for c in range(2):
        @pl.when(cid == c)
        def _():
            for p in range(2):
                cb = 2 * c + p
                h0 = 2 * cb
                h1 = h0 + 1
                cb_off = cb * NP
                for half in range(2):
                    lo = half * NPH
                    pltpu.sync_copy(
                        pcnt.at[pl.ds((half * NT + sid) * 128, 128)], cv)
                    cnt = cv[pl.ds(0, 16)][0]
                    nch = lax.shift_right_logical(cnt + 127, 7)
                    list_off = (half * NT + sid) * CAP

                    def _zr(e, c_):
                        for q in range(8):
                            rows[e, pl.ds(q * 16, 16)] = zf16
                        return c_
                    lax.fori_loop(0, B, _zr, 0)
                    for i in range(3):
                        pltpu.sync_copy(
                            rows,
                            accs.at[pl.ds(sid * (NPH // NT) + i * B, B)])

                    def _zd(t, c_):
                        sl = pl.ds(t * 16, 16)
                        den_h0[sl] = zf16
                        den_h1[sl] = zf16
                        return c_
                    lax.fori_loop(0, (NPH + 32) // 16, _zd, 0)
                    plsc.subcore_barrier()
                    amvec = amv[pl.ds(0, 16)]

                    def _slab(s6, c_):
                        pltpu.sync_copy(
                            plists.at[pl.ds(list_off + s6 * (CS * B),
                                            CS * B)], slab_v)
                        jmax = jnp.clip(nch - s6 * CS, 0, CS)

                        def _chunk(j, cc_):
                            def _off(k, c3_):
                                sl = pl.ds(k * 16, 16)
                                v16 = slab_v[pl.ds(j * B + k * 16, 16)]
                                d16 = lax.shift_right_logical(v16, 14)
                                s16 = v16 & 16383
                                tmp_s[sl] = s16
                                tmp_dr[sl] = d16
                                tmp_idx[sl] = s16 + cb_off
                                tmp_dl[sl] = d16 - lo
                                return c3_
                            lax.fori_loop(0, 8, _off, 0)
                            c1 = pltpu.async_copy(
                                h4f.at[tmp_idx], rows, sem1)
                            c2 = pltpu.async_copy(
                                lsh.at[tmp_s], lsbuf, sem2)
                            c3 = pltpu.async_copy(
                                ldh.at[tmp_dr], ldbuf, sem3)
                            c1.wait()
                            c2.wait()
                            c3.wait()

                            def _inner(k, c3_):
                                dcur = tmp_dl[pl.ds(k * 16, 16)]
                                for i in range(16):
                                    e = k * 16 + i
                                    srow = lsbuf[e, pl.ds(0, 16)]
                                    drow = ldbuf[e, pl.ds(0, 16)]
                                    al = srow + drow
                                    al = jnp.where(al >= 0, al, 0.2 * al)
                                    mv = drow + amvec
                                    mv = jnp.where(mv >= 0, mv, 0.2 * mv)
                                    exv = jnp.exp(al - mv)
                                    m0 = exv[h0]
                                    m1 = exv[h1]
                                    for q in range(8):
                                        sl = pl.ds(q * 16, 16)
                                        mm = m0 if q < 4 else m1
                                        rows[e, sl] = rows[e, sl] * mm
                                    nl = dcur[i]
                                    dv0 = den_h0[pl.ds(nl, 16)]
                                    den_h0[pl.ds(nl, 16)] = dv0 + jnp.where(
                                        iota16 == 0, m0, 0.0)
                                    dv1 = den_h1[pl.ds(nl, 16)]
                                    den_h1[pl.ds(nl, 16)] = dv1 + jnp.where(
                                        iota16 == 0, m1, 0.0)
                                return c3_
                            lax.fori_loop(0, 8, _inner, 0)
                            pltpu.sync_copy(rows, accs.at[tmp_dl], add=True)
                            return cc_
                        lax.fori_loop(0, jmax, _chunk, 0)
                        return c_
                    lax.fori_loop(0, 8, _slab, 0)
                    so0 = ((cb * 2 + 0) * NT + sid) * NP + lo
                    so1 = ((cb * 2 + 1) * NT + sid) * NP + lo
                    pltpu.sync_copy(den_h0.at[pl.ds(0, NPH)],
                                    denstage.at[pl.ds(so0, NPH)])
                    pltpu.sync_copy(den_h1.at[pl.ds(0, NPH)],
                                    denstage.at[pl.ds(so1, NPH)])
                    plsc.subcore_barrier()
                    st = pl.ds(sid * (NPH // NT), NPH // NT)
                    gst = pl.ds(lo + sid * (NPH // NT), NPH // NT)
                    pltpu.sync_copy(accs.at[st], acc_out.at[cb, gst])

                noff = sid * (NP // NT)
                for hh in range(2):
                    def _za(t, c_):
                        dacc[pl.ds(t * 16, 16)] = zf16
                        return c_
                    lax.fori_loop(0, (NP // NT) // 16, _za, 0)
                    for t in range(NT):
                        pltpu.sync_copy(
                            denstage.at[pl.ds(
                                ((cb * 2 + hh) * NT + t) * NP + noff,
                                NP // NT)], dtmp)

                        def _ad(q, c_):
                            sl = pl.ds(q * 16, 16)
                            dacc[sl] = dacc[sl] + dtmp[sl]
                            return c_
                        lax.fori_loop(0, (NP // NT) // 16, _ad, 0)
                    pltpu.sync_copy(
                        dacc,
                        den_out.at[pl.ds((cb * 2 + hh) * NP + noff,
                                         NP // NT)])


def _sc_edge(plists, pcnt, h4f, lsh, ldh, amaxh):
    mesh = plsc.VectorSubcoreMesh(core_axis_name="c", subcore_axis_name="s")
    kern = pl.kernel(
        _sc_edge_body,
        out_type=(jax.ShapeDtypeStruct((4, NP, 128), jnp.float32),
                  jax.ShapeDtypeStruct((4 * 2 * NP,), jnp.float32),
                  jax.ShapeDtypeStruct((4 * 2 * NT * NP,), jnp.float32)),
        mesh=mesh,
        scratch_types=[
            pltpu.VMEM((CS * B,), jnp.int32),
            pltpu.VMEM((128,), jnp.int32),
            pltpu.VMEM((B,), jnp.int32),
            pltpu.VMEM((B,), jnp.int32),
            pltpu.VMEM((B,), jnp.int32),
            pltpu.VMEM((B,), jnp.int32),
            pltpu.VMEM((B, 128), jnp.float32),
            pltpu.VMEM((B, 128), jnp.float32),
            pltpu.VMEM((B, 128), jnp.float32),
            pltpu.VMEM((NPH + 144,), jnp.float32),
            pltpu.VMEM((NPH + 144,), jnp.float32),
            pltpu.VMEM((NP // NT,), jnp.float32),
            pltpu.VMEM((NP // NT,), jnp.float32),
            pltpu.VMEM((128,), jnp.float32),
            pltpu.SemaphoreType.DMA,
            pltpu.SemaphoreType.DMA,
            pltpu.SemaphoreType.DMA,
            pltpu.VMEM_SHARED((NPH + 128, 128), jnp.float32),
        ],
    )
    acc4, den4, _ = kern(plists, pcnt, h4f, lsh, ldh, amaxh)
    return acc4, den4


def _sc_edge2_body(srcp, dstp, lth, amaxh,
                   o2_out, o2stage,
                   src_idx, dst_idx, ltS, ltD, pnum, pden, dtmp, dacc, amv,
                   sem1, sem2):
    cid = lax.axis_index("c")
    sid = lax.axis_index("s")
    i32 = jnp.int32
    f32 = jnp.float32
    iota16 = lax.iota(i32, 16)
    zf16 = jnp.zeros((16,), f32)
    wid = sid * 2 + cid

    pltpu.sync_copy(srcp.at[wid], src_idx)
    pltpu.sync_copy(dstp.at[wid], dst_idx)
    pltpu.sync_copy(amaxh, amv)

    def _zd(t, c_):
        sl = pl.ds(t * 16, 16)
        pnum[sl] = zf16
        pden[sl] = zf16
        return c_
    lax.fori_loop(0, (NP + 16) // 16, _zd, 0)
    amvec = amv[pl.ds(0, 16)]
    am = amvec[0]

    def _chunk(j, c_):
        c1 = pltpu.async_copy(lth.at[src_idx.at[j]], ltS, sem1)
        c2 = pltpu.async_copy(lth.at[dst_idx.at[j]], ltD, sem2)
        c1.wait()
        c2.wait()

        def _inner(k, cc_):
            dcur = dst_idx[j, pl.ds(k * 16, 16)]
            for i in range(16):
                e = k * 16 + i
                srow = ltS[e, pl.ds(0, 16)]
                drow = ltD[e, pl.ds(0, 16)]
                h2s = srow[0]
                as_ = srow[1]
                ad_ = drow[2]
                al = as_ + ad_
                al = jnp.where(al >= 0, al, 0.2 * al)
                mm = ad_ + am
                mm = jnp.where(mm >= 0, mm, 0.2 * mm)
                exv = jnp.exp(al - mm + zf16)
                n = dcur[i]
                nv = pnum[pl.ds(n, 16)]
                pnum[pl.ds(n, 16)] = nv + jnp.where(
                    iota16 == 0, exv * h2s, 0.0)
                dv = pden[pl.ds(n, 16)]
                pden[pl.ds(n, 16)] = dv + jnp.where(iota16 == 0, exv, 0.0)
            return cc_
        lax.fori_loop(0, 8, _inner, 0)
        return c_
    lax.fori_loop(0, CH // 2, _chunk, 0)
    pltpu.sync_copy(pnum.at[pl.ds(0, NP)],
                    o2stage.at[pl.ds(wid * NP, NP)])
    pltpu.sync_copy(pden.at[pl.ds(0, NP)],
                    o2stage.at[pl.ds((32 + wid) * NP, NP)])
    plsc.subcore_barrier()
    noff = sid * (NP // NT)
    for hh in range(2):
        def _za(t, c_):
            dacc[pl.ds(t * 16, 16)] = zf16
            return c_
        lax.fori_loop(0, (NP // NT) // 16, _za, 0)
        for t in range(NT):
            pltpu.sync_copy(
                o2stage.at[pl.ds((hh * 32 + t * 2 + cid) * NP + noff,
                                 NP // NT)], dtmp)

            def _ad(q, c_):
                sl = pl.ds(q * 16, 16)
                dacc[sl] = dacc[sl] + dtmp[sl]
                return c_
            lax.fori_loop(0, (NP // NT) // 16, _ad, 0)
        pltpu.sync_copy(
            dacc, o2_out.at[pl.ds((cid * 2 + hh) * NP + noff, NP // NT)])


def _sc_edge2(srcp32, dstp32, lt, amaxh):
    mesh = plsc.VectorSubcoreMesh(core_axis_name="c", subcore_axis_name="s")
    kern = pl.kernel(
        _sc_edge2_body,
        out_type=(jax.ShapeDtypeStruct((2 * 2 * NP,), jnp.float32),
                  jax.ShapeDtypeStruct((2 * 32 * NP,), jnp.float32)),
        mesh=mesh,
        scratch_types=[
            pltpu.VMEM((CH // 2, B), jnp.int32),
            pltpu.VMEM((CH // 2, B), jnp.int32),
            pltpu.VMEM((B, 128), jnp.float32),
            pltpu.VMEM((B, 128), jnp.float32),
            pltpu.VMEM((NP + 144,), jnp.float32),
            pltpu.VMEM((NP + 144,), jnp.float32),
            pltpu.VMEM((NP // NT,), jnp.float32),
            pltpu.VMEM((NP // NT,), jnp.float32),
            pltpu.VMEM((128,), jnp.float32),
            pltpu.SemaphoreType.DMA,
            pltpu.SemaphoreType.DMA,
        ],
    )
    o2, _ = kern(srcp32, dstp32, lt, amaxh)
    return o2


def kernel(x, edge_index, W0, as0, ad0, b0, W1, as1, ad1, b1, W2, as2, ad2, b2):
    xp = jnp.pad(x, ((0, NP - N), (0, 0)))
    loop = jnp.arange(N, dtype=jnp.int32)
    npad = EP - E_REAL
    padv = N + (jnp.arange(npad, dtype=jnp.int32) % (NP - N))
    srcp = jnp.concatenate([edge_index[0], loop, padv]).reshape(NT, CH, B)
    dstp = jnp.concatenate([edge_index[1], loop, padv]).reshape(NT, CH, B)
    srcp32 = srcp.reshape(32, CH // 2, B)
    dstp32 = dstp.reshape(32, CH // 2, B)

    plists, pcnt = _sc_prep(srcp, dstp)
    h4, ls0, ld0, amax = _pre(xp, W0, as0, ad0)
    amaxh = jnp.pad(amax[:, 0], (0, 120))
    acc4, den4 = _sc_edge(plists, pcnt, h4.reshape(4 * NP, 128),
                          ls0, ld0, amaxh)
    den4 = den4.reshape(4, 2, NP)

    h4b, ls1, ld1, amaxb = _mid(acc4, den4, b0.reshape(1, 512), W1, as1, ad1)
    amaxbh = jnp.pad(amaxb[:, 0], (0, 120))
    acc4b, den4b = _sc_edge(plists, pcnt, h4b.reshape(4 * NP, 128),
                            ls1, ld1, amaxbh)
    den4b = den4b.reshape(4, 2, NP)

    W2p = jnp.pad(W2, ((0, 0), (0, 127)))
    lt, amax2 = _mid2(acc4b, den4b, b1.reshape(1, 512), W2p, as2, ad2)
    amax2h = jnp.pad(amax2[:1, 0], (0, 127))
    o2 = _sc_edge2(srcp32, dstp32, lt, amax2h).reshape(2, 2, NP)
    outp = _fin(o2, b2.reshape(1, 1))
    return outp.reshape(NP)[:N]
